# baseline (speedup 1.0000x reference)
import jax
import jax.numpy as jnp
from jax.experimental import pallas as pl
from jax.experimental.pallas import tpu as pltpu

NCHUNK = 8


def _body(x_hbm, t_ref, o_hbm, sems):
    rows = x_hbm.shape[0] // NCHUNK
    for c in range(NCHUNK):
        pltpu.make_async_copy(
            x_hbm.at[pl.ds(c * rows, rows)], o_hbm.at[pl.ds(c * rows, rows)],
            sems.at[c],
        ).start()
    for c in range(NCHUNK):
        pltpu.make_async_copy(
            x_hbm.at[pl.ds(c * rows, rows)], o_hbm.at[pl.ds(c * rows, rows)],
            sems.at[c],
        ).wait()


def kernel(x, table, num_people=100):
    return pl.pallas_call(
        _body,
        in_specs=[
            pl.BlockSpec(memory_space=pltpu.MemorySpace.HBM),
            pl.BlockSpec(memory_space=pltpu.MemorySpace.VMEM),
        ],
        out_specs=pl.BlockSpec(memory_space=pltpu.MemorySpace.HBM),
        out_shape=jax.ShapeDtypeStruct(x.shape, x.dtype),
        scratch_shapes=[pltpu.SemaphoreType.DMA((NCHUNK,))],
    )(x, table)


# speedup vs baseline: 6.8792x; 6.8792x over previous
import functools

import jax
import jax.numpy as jnp
from jax import lax
from jax.experimental import pallas as pl
from jax.experimental.pallas import tpu as pltpu
from jax.experimental.pallas import tpu_sc as plsc

SEQ_LEN = 10
MAX_NORM = 1.0
L = 16


def _renorm_body(t_ref, o_ref):
    emb = t_ref[...]
    ns = jnp.sum(emb * emb, axis=1, keepdims=True)
    norm = jnp.sqrt(ns)
    scale = jnp.where(norm > MAX_NORM, MAX_NORM / (norm + 1e-7), 1.0)
    o_ref[...] = emb * scale


def kernel(x, table, num_people=100):
    n_rows, t_len, d = x.shape
    persons = n_rows // SEQ_LEN
    info = plsc.get_sparse_core_info()
    nw = info.num_cores * info.num_subcores
    rows_max = -(-n_rows // nw)
    nk = d // L

    scaled = pl.pallas_call(
        _renorm_body,
        grid=(1,),
        in_specs=[pl.BlockSpec((persons, d), lambda i: (0, 0))],
        out_specs=pl.BlockSpec((persons, d), lambda i: (0, 0)),
        out_shape=jax.ShapeDtypeStruct((persons, d), jnp.float32),
    )(table[:persons])

    mesh = plsc.VectorSubcoreMesh(core_axis_name="c", subcore_axis_name="s")

    @functools.partial(
        pl.kernel,
        mesh=mesh,
        out_type=jax.ShapeDtypeStruct(x.shape, x.dtype),
        compiler_params=pltpu.CompilerParams(needs_layout_passes=False),
        scratch_types=[
            pltpu.VMEM((t_len, d), jnp.float32),
            pltpu.VMEM((persons, d), jnp.float32),
        ],
    )
    def sc_kernel(x_hbm, e_hbm, o_hbm, xbuf, embbuf):
        wid = lax.axis_index("s") * info.num_cores + lax.axis_index("c")
        r0 = wid * n_rows // nw
        r1 = (wid + 1) * n_rows // nw

        pltpu.sync_copy(e_hbm, embbuf)

        for i in range(rows_max):
            r = jnp.minimum(r0 + i, r1 - 1)
            pltpu.sync_copy(x_hbm.at[r], xbuf)

            p = r // SEQ_LEN
            ek = [embbuf[p, pl.ds(k * L, L)] for k in range(nk)]

            def trow(t, _):
                for k in range(nk):
                    sl = pl.ds(k * L, L)
                    xbuf[t, sl] = xbuf[t, sl] + ek[k]
                return 0

            lax.fori_loop(0, t_len, trow, 0)
            pltpu.sync_copy(xbuf, o_hbm.at[r])

    return sc_kernel(x, scaled)


# speedup vs baseline: 8.4072x; 1.2221x over previous
import functools

import jax
import jax.numpy as jnp
from jax import lax
from jax.experimental import pallas as pl
from jax.experimental.pallas import tpu as pltpu
from jax.experimental.pallas import tpu_sc as plsc

SEQ_LEN = 10
MAX_NORM = 1.0
L = 16


def _renorm_body(t_ref, o_ref):
    emb = t_ref[...]
    ns = jnp.sum(emb * emb, axis=1, keepdims=True)
    norm = jnp.sqrt(ns)
    scale = jnp.where(norm > MAX_NORM, MAX_NORM / (norm + 1e-7), 1.0)
    o_ref[...] = emb * scale


def kernel(x, table, num_people=100):
    n_rows, t_len, d = x.shape
    persons = n_rows // SEQ_LEN
    info = plsc.get_sparse_core_info()
    nw = info.num_cores * info.num_subcores
    rows_max = -(-n_rows // nw)
    nk = d // L

    scaled = pl.pallas_call(
        _renorm_body,
        grid=(1,),
        in_specs=[pl.BlockSpec((persons, d), lambda i: (0, 0))],
        out_specs=pl.BlockSpec((persons, d), lambda i: (0, 0)),
        out_shape=jax.ShapeDtypeStruct((persons, d), jnp.float32),
    )(table[:persons])

    mesh = plsc.VectorSubcoreMesh(core_axis_name="c", subcore_axis_name="s")

    @functools.partial(
        pl.kernel,
        mesh=mesh,
        out_type=jax.ShapeDtypeStruct(x.shape, x.dtype),
        compiler_params=pltpu.CompilerParams(needs_layout_passes=False),
        scratch_types=[
            pltpu.VMEM((t_len, d), jnp.float32),
            pltpu.VMEM((t_len, d), jnp.float32),
            pltpu.VMEM((persons, d), jnp.float32),
            pltpu.SemaphoreType.DMA,
            pltpu.SemaphoreType.DMA,
            pltpu.SemaphoreType.DMA,
            pltpu.SemaphoreType.DMA,
        ],
    )
    def sc_kernel(x_hbm, e_hbm, o_hbm, xbuf0, xbuf1, embbuf,
                  isem0, isem1, osem0, osem1):
        xbufs = (xbuf0, xbuf1)
        isems = (isem0, isem1)
        osems = (osem0, osem1)
        wid = lax.axis_index("s") * info.num_cores + lax.axis_index("c")
        r0 = wid * n_rows // nw
        r1 = (wid + 1) * n_rows // nw

        pltpu.sync_copy(e_hbm, embbuf)

        def row_at(i):
            return jnp.minimum(r0 + i, r1 - 1)

        def copy_in(i, slot):
            return pltpu.make_async_copy(
                x_hbm.at[row_at(i)], xbufs[slot], isems[slot])

        def copy_out(i, slot):
            return pltpu.make_async_copy(
                xbufs[slot], o_hbm.at[row_at(i)], osems[slot])

        copy_in(0, 0).start()
        for i in range(rows_max):
            slot = i & 1
            xb = xbufs[slot]
            if i + 1 < rows_max:
                if i >= 1:
                    copy_out(i - 1, slot ^ 1).wait()
                copy_in(i + 1, slot ^ 1).start()
            copy_in(i, slot).wait()

            p = row_at(i) // SEQ_LEN
            ek = [embbuf[p, pl.ds(k * L, L)] for k in range(nk)]

            def trow(t, _):
                for k in range(nk):
                    sl = pl.ds(k * L, L)
                    xb[t, sl] = xb[t, sl] + ek[k]
                return 0

            lax.fori_loop(0, t_len, trow, 0)
            copy_out(i, slot).start()

        copy_out(rows_max - 2, rows_max & 1).wait()
        copy_out(rows_max - 1, (rows_max - 1) & 1).wait()

    return sc_kernel(x, scaled)


# speedup vs baseline: 9.0117x; 1.0719x over previous
import functools

import jax
import jax.numpy as jnp
from jax import lax
from jax.experimental import pallas as pl
from jax.experimental.pallas import tpu as pltpu
from jax.experimental.pallas import tpu_sc as plsc

SEQ_LEN = 10
MAX_NORM = 1.0
L = 16
CROWS = 5


def _renorm_body(t_ref, o_ref):
    emb = t_ref[...]
    ns = jnp.sum(emb * emb, axis=1, keepdims=True)
    norm = jnp.sqrt(ns)
    scale = jnp.where(norm > MAX_NORM, MAX_NORM / (norm + 1e-7), 1.0)
    o_ref[...] = emb * scale


def kernel(x, table, num_people=100):
    n_rows, t_len, d = x.shape
    persons = n_rows // SEQ_LEN
    info = plsc.get_sparse_core_info()
    nw = info.num_cores * info.num_subcores
    n_chunks = n_rows // CROWS
    cmax = -(-n_chunks // nw)
    nk = d // L

    scaled = pl.pallas_call(
        _renorm_body,
        grid=(1,),
        in_specs=[pl.BlockSpec((persons, d), lambda i: (0, 0))],
        out_specs=pl.BlockSpec((persons, d), lambda i: (0, 0)),
        out_shape=jax.ShapeDtypeStruct((persons, d), jnp.float32),
    )(table[:persons])

    mesh = plsc.VectorSubcoreMesh(core_axis_name="c", subcore_axis_name="s")

    @functools.partial(
        pl.kernel,
        mesh=mesh,
        out_type=jax.ShapeDtypeStruct(x.shape, x.dtype),
        compiler_params=pltpu.CompilerParams(needs_layout_passes=False),
        scratch_types=[
            pltpu.VMEM((CROWS * t_len, d), jnp.float32),
            pltpu.VMEM((CROWS * t_len, d), jnp.float32),
            pltpu.VMEM((persons, d), jnp.float32),
            pltpu.SemaphoreType.DMA,
            pltpu.SemaphoreType.DMA,
            pltpu.SemaphoreType.DMA,
            pltpu.SemaphoreType.DMA,
        ],
    )
    def sc_kernel(x_hbm, e_hbm, o_hbm, xbuf0, xbuf1, embbuf,
                  isem0, isem1, osem0, osem1):
        xbufs = (xbuf0, xbuf1)
        isems = (isem0, isem1)
        osems = (osem0, osem1)
        wid = lax.axis_index("s") * info.num_cores + lax.axis_index("c")
        c0 = wid * n_chunks // nw
        c1 = (wid + 1) * n_chunks // nw

        pltpu.sync_copy(e_hbm, embbuf)

        def chunk_at(i):
            return jnp.minimum(c0 + i, c1 - 1)

        def copy_in(i, slot, j):
            return pltpu.make_async_copy(
                x_hbm.at[chunk_at(i) * CROWS + j],
                xbufs[slot].at[pl.ds(j * t_len, t_len)], isems[slot])

        def copy_out(i, slot, j):
            return pltpu.make_async_copy(
                xbufs[slot].at[pl.ds(j * t_len, t_len)],
                o_hbm.at[chunk_at(i) * CROWS + j], osems[slot])

        for j in range(CROWS):
            copy_in(0, 0, j).start()
        for i in range(cmax):
            slot = i & 1
            xb = xbufs[slot]
            if i + 1 < cmax:
                if i >= 1:
                    for j in range(CROWS):
                        copy_out(i - 1, slot ^ 1, j).wait()
                for j in range(CROWS):
                    copy_in(i + 1, slot ^ 1, j).start()
            for j in range(CROWS):
                copy_in(i, slot, j).wait()

            p = chunk_at(i) * CROWS // SEQ_LEN
            ek = [embbuf[p, pl.ds(k * L, L)] for k in range(nk)]

            @plsc.parallel_loop(0, CROWS * t_len, unroll=4)
            def trow(jt):
                for k in range(nk):
                    sl = pl.ds(k * L, L)
                    xb[jt, sl] = xb[jt, sl] + ek[k]

            for j in range(CROWS):
                copy_out(i, slot, j).start()

        for j in range(CROWS):
            copy_out(cmax - 2, cmax & 1, j).wait()
            copy_out(cmax - 1, (cmax - 1) & 1, j).wait()

    return sc_kernel(x, scaled)


# speedup vs baseline: 9.0432x; 1.0035x over previous
import functools

import jax
import jax.numpy as jnp
from jax import lax
from jax.experimental import pallas as pl
from jax.experimental.pallas import tpu as pltpu
from jax.experimental.pallas import tpu_sc as plsc

SEQ_LEN = 10
MAX_NORM = 1.0
L = 16
CROWS = 5


def _renorm_body(t_ref, o_ref):
    emb = t_ref[...]
    ns = jnp.sum(emb * emb, axis=1, keepdims=True)
    norm = jnp.sqrt(ns)
    scale = jnp.where(norm > MAX_NORM, MAX_NORM / (norm + 1e-7), 1.0)
    o_ref[...] = emb * scale


def kernel(x, table, num_people=100):
    n_rows, t_len, d = x.shape
    persons = n_rows // SEQ_LEN
    info = plsc.get_sparse_core_info()
    nw = info.num_cores * info.num_subcores
    n_chunks = n_rows // CROWS
    cmax = -(-n_chunks // nw)
    nk = d // L

    scaled = pl.pallas_call(
        _renorm_body,
        grid=(1,),
        in_specs=[pl.BlockSpec((persons, d), lambda i: (0, 0))],
        out_specs=pl.BlockSpec((persons, d), lambda i: (0, 0)),
        out_shape=jax.ShapeDtypeStruct((persons, d), jnp.float32),
    )(table[:persons])

    mesh = plsc.VectorSubcoreMesh(core_axis_name="c", subcore_axis_name="s")

    @functools.partial(
        pl.kernel,
        mesh=mesh,
        out_type=jax.ShapeDtypeStruct(x.shape, x.dtype),
        compiler_params=pltpu.CompilerParams(needs_layout_passes=False),
        scratch_types=[
            pltpu.VMEM((CROWS * t_len, d), jnp.float32),
            pltpu.VMEM((CROWS * t_len, d), jnp.float32),
            pltpu.VMEM((CROWS * t_len, d), jnp.float32),
            pltpu.VMEM((persons, d), jnp.float32),
            pltpu.SemaphoreType.DMA,
            pltpu.SemaphoreType.DMA,
            pltpu.SemaphoreType.DMA,
            pltpu.SemaphoreType.DMA,
            pltpu.SemaphoreType.DMA,
            pltpu.SemaphoreType.DMA,
        ],
    )
    def sc_kernel(x_hbm, e_hbm, o_hbm, xbuf0, xbuf1, xbuf2, embbuf,
                  isem0, isem1, isem2, osem0, osem1, osem2):
        xbufs = (xbuf0, xbuf1, xbuf2)
        isems = (isem0, isem1, isem2)
        osems = (osem0, osem1, osem2)
        ns = 3
        wid = lax.axis_index("s") * info.num_cores + lax.axis_index("c")
        c0 = wid * n_chunks // nw
        c1 = (wid + 1) * n_chunks // nw

        pltpu.sync_copy(e_hbm, embbuf)

        def chunk_at(i):
            return jnp.minimum(c0 + i, c1 - 1)

        def copy_in(i, slot, j):
            return pltpu.make_async_copy(
                x_hbm.at[chunk_at(i) * CROWS + j],
                xbufs[slot].at[pl.ds(j * t_len, t_len)], isems[slot])

        def copy_out(i, slot, j):
            return pltpu.make_async_copy(
                xbufs[slot].at[pl.ds(j * t_len, t_len)],
                o_hbm.at[chunk_at(i) * CROWS + j], osems[slot])

        for s in range(ns - 1):
            for j in range(CROWS):
                copy_in(s, s, j).start()
        for i in range(cmax):
            slot = i % ns
            xb = xbufs[slot]
            nxt = i + ns - 1
            if nxt < cmax:
                pslot = nxt % ns
                if nxt >= ns:
                    for j in range(CROWS):
                        copy_out(nxt - ns, pslot, j).wait()
                for j in range(CROWS):
                    copy_in(nxt, pslot, j).start()
            for j in range(CROWS):
                copy_in(i, slot, j).wait()

            p = chunk_at(i) * CROWS // SEQ_LEN
            ek = [embbuf[p, pl.ds(k * L, L)] for k in range(nk)]

            @plsc.parallel_loop(0, CROWS * t_len, unroll=8)
            def trow(jt):
                for k in range(nk):
                    sl = pl.ds(k * L, L)
                    xb[jt, sl] = xb[jt, sl] + ek[k]

            for j in range(CROWS):
                copy_out(i, slot, j).start()

        for i in range(cmax - ns, cmax):
            for j in range(CROWS):
                copy_out(i, i % ns, j).wait()

    return sc_kernel(x, scaled)
